# baseline (speedup 1.0000x reference)
import functools

import jax
import jax.numpy as jnp
from jax import lax
from jax.experimental import pallas as pl
from jax.experimental.pallas import tpu as pltpu
from jax.experimental.pallas import tpu_sc as plsc

_NW = 32
_CH = 800
_NB = 4


def _gather_body(per_w, ids_hbm, table_hbm, out_hbm,
                 idx_v, rows, gsems, osems):
    ch = _CH
    n_chunks = per_w // ch
    wid = lax.axis_index("s") * 2 + lax.axis_index("c")
    base = wid * per_w
    pltpu.sync_copy(ids_hbm.at[pl.ds(base, per_w)], idx_v)

    def g_copy(g, k):
        return pltpu.make_async_copy(
            table_hbm.at[idx_v.at[pl.ds(g * ch, ch)]], rows[k], gsems[k])

    def s_copy(g, k):
        return pltpu.make_async_copy(
            rows[k], out_hbm.at[pl.ds(base + g * ch, ch)], osems[k])

    g_copy(0, 0).start()
    g_copy(1, 1).start()

    def body(i, carry):
        a = _NB * i
        for k in range(_NB):
            kp = (k + 2) % _NB
            g_copy(a + k, k).wait()
            s_copy(a + k, k).start()

            @pl.when(a + k + 2 < n_chunks)
            def _():
                @pl.when(a + k >= 2)
                def _():
                    s_copy(a + k - 2, kp).wait()

                g_copy(a + k + 2, kp).start()

        return carry

    lax.fori_loop(0, n_chunks // _NB, body, 0, unroll=False)
    for k in range(_NB):
        s_copy(0, k).wait()


def _tpose_body(tt_ref, o_ref):
    w = jnp.reshape(tt_ref[...].T, (128, 4, 32))
    o_ref[...] = jnp.concatenate([w[:, q, :] for q in range(4)], axis=1)


def _pack_table(table):
    v, d = table.shape
    tt = table.T
    nblk = pl.cdiv(v, 512)
    return pl.pallas_call(
        _tpose_body,
        grid=(nblk,),
        in_specs=[pl.BlockSpec((d, 512), lambda i: (0, i))],
        out_specs=pl.BlockSpec((128, 128), lambda i: (i, 0)),
        out_shape=jax.ShapeDtypeStruct((v // 4, 4 * d), jnp.float32),
    )(tt)


def kernel(token_ids, table):
    b, s = token_ids.shape
    v, d = table.shape
    n = b * s
    assert n % (_NW * _NB * _CH) == 0
    per_w = n // _NW

    flat_ids = token_ids.reshape(n).astype(jnp.int32)
    table_rm = jnp.reshape(_pack_table(table), (v, d))
    mesh = plsc.VectorSubcoreMesh(core_axis_name="c", subcore_axis_name="s")
    k = pl.kernel(
        functools.partial(_gather_body, per_w),
        out_type=jax.ShapeDtypeStruct((n, d), jnp.float32),
        mesh=mesh,
        scratch_types=[
            pltpu.VMEM((per_w,), jnp.int32),
            [pltpu.VMEM((_CH, d), jnp.float32) for _ in range(_NB)],
            [pltpu.SemaphoreType.DMA for _ in range(_NB)],
            [pltpu.SemaphoreType.DMA for _ in range(_NB)],
        ],
        compiler_params=pltpu.CompilerParams(use_tc_tiling_on_sc=False),
    )
    out = k(flat_ids, table_rm)
    return out.reshape(b, s, d)
